# baseline (speedup 1.0000x reference)
import math

import jax
import jax.numpy as jnp
from jax.experimental import pallas as pl

PATCH = 16
HEADS = 8

_DN0 = (((0,), (0,)), ((), ()))
_F32 = jnp.float32
_BF16 = jnp.bfloat16


def _pool_kernel(x_ref, out_ref):
    xb = x_ref[0]
    Cc, P, Wd = xb.shape
    gw = Wd // P
    rs = jnp.sum(xb, axis=1)
    hi = rs.astype(_BF16)
    r1 = rs - hi.astype(_F32)
    lo = r1.astype(_BF16)
    lo2 = (r1 - lo.astype(_F32)).astype(_BF16)
    sel = (jax.lax.broadcasted_iota(jnp.int32, (Wd, gw), 0) // P ==
           jax.lax.broadcasted_iota(jnp.int32, (Wd, gw), 1)).astype(_BF16)
    m = (jnp.dot(hi, sel, preferred_element_type=_F32) +
         (jnp.dot(lo, sel, preferred_element_type=_F32) +
          jnp.dot(lo2, sel, preferred_element_type=_F32)))
    out_ref[0, 0] = m * (1.0 / (P * P))


def _assign_kernel(x_ref, sp_ref, wk_ref, wv_ref, spk_ref, spv_ref, cnt_ref):
    p = pl.program_id(1)
    xf = x_ref[0]
    xb = xf.astype(_BF16)
    spb = sp_ref[0]
    kb = jax.lax.dot_general(wk_ref[...], xb, _DN0,
                             preferred_element_type=_F32).astype(_BF16)
    vb = jax.lax.dot_general(wv_ref[...], xf, _DN0, preferred_element_type=_F32,
                             precision=jax.lax.Precision.HIGHEST)
    sims = jax.lax.dot_general(spb, xf, _DN0, preferred_element_type=_F32)
    m = jnp.max(sims, axis=0, keepdims=True)
    oh = (sims == m).astype(_F32)
    dn_pp = (((1,), (1,)), ((), ()))
    spk_c = jax.lax.dot_general(kb, oh.astype(_BF16), dn_pp,
                                preferred_element_type=_F32)
    spv_c = jax.lax.dot_general(vb, oh, dn_pp, preferred_element_type=_F32,
                                precision=jax.lax.Precision.HIGHEST)
    cnt_c = jnp.sum(oh, axis=1, keepdims=True)

    @pl.when(p == 0)
    def _():
        spk_ref[0] = spk_c
        spv_ref[0] = spv_c
        cnt_ref[0] = cnt_c

    @pl.when(p != 0)
    def _():
        spk_ref[0] += spk_c
        spv_ref[0] += spv_c
        cnt_ref[0] += cnt_c


def _attn_kernel(x_ref, wq_ref, spk_ref, spv_ref, cnt_ref, wo_ref, out_ref):
    xb = x_ref[0].astype(_BF16)
    Cc, Pb = xb.shape
    S = spk_ref.shape[2]
    dh = Cc // HEADS
    qb = jax.lax.dot_general(wq_ref[...], xb, _DN0,
                             preferred_element_type=_F32).astype(_BF16)
    inv = (1.0 / jnp.maximum(cnt_ref[0], 1.0)).reshape(1, S)
    km = (spk_ref[0] * (inv * (1.0 / math.sqrt(dh)))).astype(_BF16)
    vm = spv_ref[0] * inv
    qh = qb.reshape(HEADS, dh, Pb)
    kh = km.reshape(HEADS, dh, S)
    vh = vm.reshape(HEADS, dh, S)
    dn = (((1,), (1,)), ((0,), (0,)))
    logits = jax.lax.dot_general(kh, qh, dn, preferred_element_type=_F32)
    e = jnp.exp(logits)
    denom = jnp.sum(e, axis=1, keepdims=True)
    dn2 = (((2,), (1,)), ((0,), (0,)))
    ctx = jax.lax.dot_general(vh, e, dn2, preferred_element_type=_F32,
                              precision=jax.lax.Precision.HIGHEST)
    ctx = (ctx * (1.0 / denom)).reshape(Cc, Pb)
    out_ref[0] = jax.lax.dot_general(wo_ref[...], ctx, _DN0,
                                     preferred_element_type=_F32,
                                     precision=jax.lax.Precision.HIGHEST)


def kernel(x, Wq, Wk, Wv, Wo):
    B_, C_, H_, W_ = x.shape
    GH, GW = H_ // PATCH, W_ // PATCH
    S = GH * GW
    HWp = H_ * W_
    Pb = 1792 if HWp % 1792 == 0 else HWp
    NP = HWp // Pb
    xp = x.reshape(B_, C_, HWp)
    wqb = Wq.astype(_BF16)
    wkb = Wk.astype(_BF16)

    pooled = pl.pallas_call(
        _pool_kernel,
        grid=(B_, GH),
        in_specs=[pl.BlockSpec((1, C_, PATCH, W_), lambda b, g: (b, 0, g, 0))],
        out_specs=pl.BlockSpec((1, 1, C_, GW), lambda b, g: (b, g, 0, 0)),
        out_shape=jax.ShapeDtypeStruct((B_, GH, C_, GW), _F32),
    )(x)
    sp = pooled.transpose(0, 2, 1, 3).reshape(B_, C_, S)

    spk, spv, cnt = pl.pallas_call(
        _assign_kernel,
        grid=(B_, NP),
        in_specs=[
            pl.BlockSpec((1, C_, Pb), lambda b, p: (b, 0, p)),
            pl.BlockSpec((1, C_, S), lambda b, p: (b, 0, 0)),
            pl.BlockSpec((C_, C_), lambda b, p: (0, 0)),
            pl.BlockSpec((C_, C_), lambda b, p: (0, 0)),
        ],
        out_specs=[
            pl.BlockSpec((1, C_, S), lambda b, p: (b, 0, 0)),
            pl.BlockSpec((1, C_, S), lambda b, p: (b, 0, 0)),
            pl.BlockSpec((1, S, 1), lambda b, p: (b, 0, 0)),
        ],
        out_shape=[
            jax.ShapeDtypeStruct((B_, C_, S), _F32),
            jax.ShapeDtypeStruct((B_, C_, S), _F32),
            jax.ShapeDtypeStruct((B_, S, 1), _F32),
        ],
    )(xp, sp, wkb, Wv)

    out = pl.pallas_call(
        _attn_kernel,
        grid=(B_, NP),
        in_specs=[
            pl.BlockSpec((1, C_, Pb), lambda b, p: (b, 0, p)),
            pl.BlockSpec((C_, C_), lambda b, p: (0, 0)),
            pl.BlockSpec((1, C_, S), lambda b, p: (b, 0, 0)),
            pl.BlockSpec((1, C_, S), lambda b, p: (b, 0, 0)),
            pl.BlockSpec((1, S, 1), lambda b, p: (b, 0, 0)),
            pl.BlockSpec((C_, C_), lambda b, p: (0, 0)),
        ],
        out_specs=pl.BlockSpec((1, C_, Pb), lambda b, p: (b, 0, p)),
        out_shape=jax.ShapeDtypeStruct((B_, C_, HWp), _F32),
    )(xp, wqb, spk, spv, cnt, Wo)

    return out.reshape(B_, C_, H_, W_)


# speedup vs baseline: 2.0719x; 2.0719x over previous
import math

import jax
import jax.numpy as jnp
from jax.experimental import pallas as pl

PATCH = 16
HEADS = 8

_DN0 = (((0,), (0,)), ((), ()))
_F32 = jnp.float32
_BF16 = jnp.bfloat16


def _pool_kernel(x_ref, out_ref):
    xb = x_ref[0]
    Cc, P, Wd = xb.shape
    gw = Wd // P
    rs = jnp.sum(xb, axis=1)
    hi = rs.astype(_BF16)
    r1 = rs - hi.astype(_F32)
    lo = r1.astype(_BF16)
    lo2 = (r1 - lo.astype(_F32)).astype(_BF16)
    sel = (jax.lax.broadcasted_iota(jnp.int32, (Wd, gw), 0) // P ==
           jax.lax.broadcasted_iota(jnp.int32, (Wd, gw), 1)).astype(_BF16)
    m = (jnp.dot(hi, sel, preferred_element_type=_F32) +
         (jnp.dot(lo, sel, preferred_element_type=_F32) +
          jnp.dot(lo2, sel, preferred_element_type=_F32)))
    out_ref[0, 0] = m * (1.0 / (P * P))


def _assign_kernel(x_ref, sp_ref, wk_ref, wv_ref, spk_ref, spv_ref, cnt_ref):
    p = pl.program_id(1)
    xf = x_ref[0]
    xb = xf.astype(_BF16)
    spb = sp_ref[0]
    kb = jax.lax.dot_general(wk_ref[...], xb, _DN0,
                             preferred_element_type=_F32).astype(_BF16)
    vb = jax.lax.dot_general(wv_ref[...], xf, _DN0, preferred_element_type=_F32)
    sims = jax.lax.dot_general(spb, xf, _DN0, preferred_element_type=_F32)
    m = jnp.max(sims, axis=0, keepdims=True)
    oh = (sims == m).astype(_F32)
    dn_pp = (((1,), (1,)), ((), ()))
    spk_c = jax.lax.dot_general(kb, oh.astype(_BF16), dn_pp,
                                preferred_element_type=_F32)
    spv_c = jax.lax.dot_general(vb, oh, dn_pp, preferred_element_type=_F32)
    cnt_c = jnp.sum(oh, axis=1, keepdims=True)

    @pl.when(p == 0)
    def _():
        spk_ref[0] = spk_c
        spv_ref[0] = spv_c
        cnt_ref[0] = cnt_c

    @pl.when(p != 0)
    def _():
        spk_ref[0] += spk_c
        spv_ref[0] += spv_c
        cnt_ref[0] += cnt_c


def _attn_kernel(x_ref, wq_ref, spk_ref, spv_ref, cnt_ref, wo_ref, out_ref):
    xb = x_ref[0].astype(_BF16)
    Cc, Pb = xb.shape
    S = spk_ref.shape[2]
    dh = Cc // HEADS
    qb = jax.lax.dot_general(wq_ref[...], xb, _DN0,
                             preferred_element_type=_F32).astype(_BF16)
    inv = (1.0 / jnp.maximum(cnt_ref[0], 1.0)).reshape(1, S)
    km = (spk_ref[0] * (inv * (1.0 / math.sqrt(dh)))).astype(_BF16)
    vm = spv_ref[0] * inv
    qh = qb.reshape(HEADS, dh, Pb)
    kh = km.reshape(HEADS, dh, S)
    vh = vm.reshape(HEADS, dh, S)
    dn = (((1,), (1,)), ((0,), (0,)))
    logits = jax.lax.dot_general(kh, qh, dn, preferred_element_type=_F32)
    e = jnp.exp(logits)
    denom = jnp.sum(e, axis=1, keepdims=True)
    dn2 = (((2,), (1,)), ((0,), (0,)))
    ctx = jax.lax.dot_general(vh, e, dn2, preferred_element_type=_F32)
    ctx = (ctx * (1.0 / denom)).reshape(Cc, Pb)
    out_ref[0] = jax.lax.dot_general(wo_ref[...], ctx, _DN0,
                                     preferred_element_type=_F32)


def kernel(x, Wq, Wk, Wv, Wo):
    B_, C_, H_, W_ = x.shape
    GH, GW = H_ // PATCH, W_ // PATCH
    S = GH * GW
    HWp = H_ * W_
    Pb = 1792 if HWp % 1792 == 0 else HWp
    NP = HWp // Pb
    xp = x.reshape(B_, C_, HWp)
    wqb = Wq.astype(_BF16)
    wkb = Wk.astype(_BF16)

    pooled = pl.pallas_call(
        _pool_kernel,
        grid=(B_, GH),
        in_specs=[pl.BlockSpec((1, C_, PATCH, W_), lambda b, g: (b, 0, g, 0))],
        out_specs=pl.BlockSpec((1, 1, C_, GW), lambda b, g: (b, g, 0, 0)),
        out_shape=jax.ShapeDtypeStruct((B_, GH, C_, GW), _F32),
    )(x)
    sp = pooled.transpose(0, 2, 1, 3).reshape(B_, C_, S)

    spk, spv, cnt = pl.pallas_call(
        _assign_kernel,
        grid=(B_, NP),
        in_specs=[
            pl.BlockSpec((1, C_, Pb), lambda b, p: (b, 0, p)),
            pl.BlockSpec((1, C_, S), lambda b, p: (b, 0, 0)),
            pl.BlockSpec((C_, C_), lambda b, p: (0, 0)),
            pl.BlockSpec((C_, C_), lambda b, p: (0, 0)),
        ],
        out_specs=[
            pl.BlockSpec((1, C_, S), lambda b, p: (b, 0, 0)),
            pl.BlockSpec((1, C_, S), lambda b, p: (b, 0, 0)),
            pl.BlockSpec((1, S, 1), lambda b, p: (b, 0, 0)),
        ],
        out_shape=[
            jax.ShapeDtypeStruct((B_, C_, S), _F32),
            jax.ShapeDtypeStruct((B_, C_, S), _F32),
            jax.ShapeDtypeStruct((B_, S, 1), _F32),
        ],
    )(xp, sp, wkb, Wv)

    out = pl.pallas_call(
        _attn_kernel,
        grid=(B_, NP),
        in_specs=[
            pl.BlockSpec((1, C_, Pb), lambda b, p: (b, 0, p)),
            pl.BlockSpec((C_, C_), lambda b, p: (0, 0)),
            pl.BlockSpec((1, C_, S), lambda b, p: (b, 0, 0)),
            pl.BlockSpec((1, C_, S), lambda b, p: (b, 0, 0)),
            pl.BlockSpec((1, S, 1), lambda b, p: (b, 0, 0)),
            pl.BlockSpec((C_, C_), lambda b, p: (0, 0)),
        ],
        out_specs=pl.BlockSpec((1, C_, Pb), lambda b, p: (b, 0, p)),
        out_shape=jax.ShapeDtypeStruct((B_, C_, HWp), _F32),
    )(xp, wqb, spk, spv, cnt, Wo)

    return out.reshape(B_, C_, H_, W_)


# speedup vs baseline: 2.1111x; 1.0190x over previous
import math

import jax
import jax.numpy as jnp
from jax.experimental import pallas as pl

PATCH = 16
HEADS = 8

_DN0 = (((0,), (0,)), ((), ()))
_F32 = jnp.float32
_BF16 = jnp.bfloat16


def _pool_kernel(x_ref, out_ref):
    xb = x_ref[0]
    Cc, P, Wd = xb.shape
    gw = Wd // P
    rs = jnp.sum(xb, axis=1)
    hi = rs.astype(_BF16)
    r1 = rs - hi.astype(_F32)
    lo = r1.astype(_BF16)
    lo2 = (r1 - lo.astype(_F32)).astype(_BF16)
    sel = (jax.lax.broadcasted_iota(jnp.int32, (Wd, gw), 0) // P ==
           jax.lax.broadcasted_iota(jnp.int32, (Wd, gw), 1)).astype(_BF16)
    m = (jnp.dot(hi, sel, preferred_element_type=_F32) +
         (jnp.dot(lo, sel, preferred_element_type=_F32) +
          jnp.dot(lo2, sel, preferred_element_type=_F32)))
    out_ref[0, 0] = m * (1.0 / (P * P))


def _assign_kernel(x_ref, sp_ref, wk_ref, wv_ref, spk_ref, spv_ref, cnt_ref):
    p = pl.program_id(1)
    xf = x_ref[0]
    xb = xf.astype(_BF16)
    spb = sp_ref[0]
    kb = jax.lax.dot_general(wk_ref[...], xb, _DN0,
                             preferred_element_type=_F32).astype(_BF16)
    vb = jax.lax.dot_general(wv_ref[...], xf, _DN0, preferred_element_type=_F32)
    sims = jax.lax.dot_general(spb, xf, _DN0, preferred_element_type=_F32)
    m = jnp.max(sims, axis=0, keepdims=True)
    oh = (sims == m).astype(_F32)
    dn_pp = (((1,), (1,)), ((), ()))
    spk_c = jax.lax.dot_general(kb, oh.astype(_BF16), dn_pp,
                                preferred_element_type=_F32)
    spv_c = jax.lax.dot_general(vb, oh, dn_pp, preferred_element_type=_F32)
    cnt_c = jnp.sum(oh, axis=1, keepdims=True)

    @pl.when(p == 0)
    def _():
        spk_ref[0] = spk_c
        spv_ref[0] = spv_c
        cnt_ref[0] = cnt_c

    @pl.when(p != 0)
    def _():
        spk_ref[0] += spk_c
        spv_ref[0] += spv_c
        cnt_ref[0] += cnt_c


def _attn_kernel(x_ref, wq_ref, spk_ref, spv_ref, cnt_ref, wo_ref, out_ref):
    xb = x_ref[0].astype(_BF16)
    Cc, Pb = xb.shape
    S = spk_ref.shape[2]
    dh = Cc // HEADS
    qb = jax.lax.dot_general(wq_ref[...], xb, _DN0,
                             preferred_element_type=_F32).astype(_BF16)
    inv = (1.0 / jnp.maximum(cnt_ref[0], 1.0)).reshape(1, S)
    km = (spk_ref[0] * (inv * (1.0 / math.sqrt(dh)))).astype(_BF16)
    vm = spv_ref[0] * inv
    qh = qb.reshape(HEADS, dh, Pb)
    kh = km.reshape(HEADS, dh, S)
    vh = vm.reshape(HEADS, dh, S)
    dn = (((1,), (1,)), ((0,), (0,)))
    logits = jax.lax.dot_general(kh, qh, dn, preferred_element_type=_F32)
    e = jnp.exp(logits)
    denom = jnp.sum(e, axis=1, keepdims=True)
    dn2 = (((2,), (1,)), ((0,), (0,)))
    ctx = jax.lax.dot_general(vh, e, dn2, preferred_element_type=_F32)
    ctx = (ctx * (1.0 / denom)).reshape(Cc, Pb)
    out_ref[0] = jax.lax.dot_general(wo_ref[...], ctx, _DN0,
                                     preferred_element_type=_F32)


def kernel(x, Wq, Wk, Wv, Wo):
    B_, C_, H_, W_ = x.shape
    GH, GW = H_ // PATCH, W_ // PATCH
    S = GH * GW
    HWp = H_ * W_
    Pb = 3584 if HWp % 3584 == 0 else HWp
    NP = HWp // Pb
    xp = x.reshape(B_, C_, HWp)
    wqb = Wq.astype(_BF16)
    wkb = Wk.astype(_BF16)

    pooled = pl.pallas_call(
        _pool_kernel,
        grid=(B_, GH),
        in_specs=[pl.BlockSpec((1, C_, PATCH, W_), lambda b, g: (b, 0, g, 0))],
        out_specs=pl.BlockSpec((1, 1, C_, GW), lambda b, g: (b, g, 0, 0)),
        out_shape=jax.ShapeDtypeStruct((B_, GH, C_, GW), _F32),
    )(x)
    sp = pooled.transpose(0, 2, 1, 3).reshape(B_, C_, S)

    spk, spv, cnt = pl.pallas_call(
        _assign_kernel,
        grid=(B_, NP),
        in_specs=[
            pl.BlockSpec((1, C_, Pb), lambda b, p: (b, 0, p)),
            pl.BlockSpec((1, C_, S), lambda b, p: (b, 0, 0)),
            pl.BlockSpec((C_, C_), lambda b, p: (0, 0)),
            pl.BlockSpec((C_, C_), lambda b, p: (0, 0)),
        ],
        out_specs=[
            pl.BlockSpec((1, C_, S), lambda b, p: (b, 0, 0)),
            pl.BlockSpec((1, C_, S), lambda b, p: (b, 0, 0)),
            pl.BlockSpec((1, S, 1), lambda b, p: (b, 0, 0)),
        ],
        out_shape=[
            jax.ShapeDtypeStruct((B_, C_, S), _F32),
            jax.ShapeDtypeStruct((B_, C_, S), _F32),
            jax.ShapeDtypeStruct((B_, S, 1), _F32),
        ],
    )(xp, sp, wkb, Wv)

    out = pl.pallas_call(
        _attn_kernel,
        grid=(B_, NP),
        in_specs=[
            pl.BlockSpec((1, C_, Pb), lambda b, p: (b, 0, p)),
            pl.BlockSpec((C_, C_), lambda b, p: (0, 0)),
            pl.BlockSpec((1, C_, S), lambda b, p: (b, 0, 0)),
            pl.BlockSpec((1, C_, S), lambda b, p: (b, 0, 0)),
            pl.BlockSpec((1, S, 1), lambda b, p: (b, 0, 0)),
            pl.BlockSpec((C_, C_), lambda b, p: (0, 0)),
        ],
        out_specs=pl.BlockSpec((1, C_, Pb), lambda b, p: (b, 0, p)),
        out_shape=jax.ShapeDtypeStruct((B_, C_, HWp), _F32),
    )(xp, wqb, spk, spv, cnt, Wo)

    return out.reshape(B_, C_, H_, W_)
